# Initial kernel scaffold; baseline (speedup 1.0000x reference)
import jax
import jax.numpy as jnp
from jax.experimental import pallas as pl


def kernel(n_feats, edge_index, Ws1, Wn1, b1, Wsm, Wnm, bm, Wsl, Wnl, bl):
    raise NotImplementedError("write your pallas kernel here")


# speedup vs baseline: 7.4269x; 7.4269x over previous
import functools

import jax
import jax.numpy as jnp
from jax import lax
from jax.experimental import pallas as pl
from jax.experimental.pallas import tpu as pltpu
from jax.experimental.pallas import tpu_sc as plsc

N = 10000
E = 320000
D = 128
H = 32

NW = 32
KB = 79
EPW = KB * 128
EPAD = NW * EPW
NPAD = 10112
RPT = NPAD // 16

ROWB = 2000
DEC_BR = 200


def _mm_body(x_ref, w_ref, o_ref):
    o_ref[...] = jnp.dot(x_ref[...], w_ref[...],
                         preferred_element_type=jnp.float32)


def _matmul(x, w):
    n, k = x.shape
    m = w.shape[1]
    return pl.pallas_call(
        _mm_body,
        grid=(n // ROWB,),
        in_specs=[pl.BlockSpec((ROWB, k), lambda i: (i, 0)),
                  pl.BlockSpec((k, m), lambda i: (0, 0))],
        out_specs=pl.BlockSpec((ROWB, m), lambda i: (i, 0)),
        out_shape=jax.ShapeDtypeStruct((n, m), jnp.float32),
    )(x, w)


def _sc_aggregate(table, srcw, dstw, zagg, zdeg, ones16, with_deg):
    mesh = plsc.VectorSubcoreMesh(core_axis_name="c", subcore_axis_name="s")

    out_type = [jax.ShapeDtypeStruct((NPAD, H), jnp.float32),
                jax.ShapeDtypeStruct((NPAD, H), jnp.float32)]
    scratch = [pltpu.VMEM((KB, 128), jnp.int32),
               pltpu.VMEM((KB, 128), jnp.int32),
               pltpu.VMEM((128, H), jnp.float32),
               pltpu.VMEM_SHARED((NPAD, H), jnp.float32),
               pltpu.SemaphoreType.DMA]
    if with_deg:
        out_type += [jax.ShapeDtypeStruct((NPAD, 16), jnp.float32),
                     jax.ShapeDtypeStruct((NPAD, 16), jnp.float32)]
        scratch += [pltpu.VMEM((128, 16), jnp.float32),
                    pltpu.VMEM_SHARED((NPAD, 16), jnp.float32)]

    def body(*refs):
        if with_deg:
            (tab_hbm, src_hbm, dst_hbm, zagg_hbm, zdeg_hbm, ones_hbm,
             agg0_hbm, agg1_hbm, deg0_hbm, deg1_hbm,
             src_v, dst_v, rows_v, sh_agg, sem, ones_v, sh_deg) = refs
        else:
            (tab_hbm, src_hbm, dst_hbm, zagg_hbm,
             agg0_hbm, agg1_hbm,
             src_v, dst_v, rows_v, sh_agg, sem) = refs
        c = lax.axis_index("c")
        s = lax.axis_index("s")
        w = s * 2 + c

        @pl.when(s == 0)
        def _():
            pltpu.sync_copy(zagg_hbm, sh_agg)
            if with_deg:
                pltpu.sync_copy(zdeg_hbm, sh_deg)

        pltpu.sync_copy(src_hbm.at[w], src_v)
        pltpu.sync_copy(dst_hbm.at[w], dst_v)
        if with_deg:
            pltpu.sync_copy(ones_hbm, ones_v)
        plsc.subcore_barrier()

        def step(j, carry):
            pltpu.async_copy(tab_hbm.at[src_v.at[j]], rows_v, sem).wait()
            pltpu.sync_copy(rows_v, sh_agg.at[dst_v.at[j]], add=True)
            if with_deg:
                pltpu.sync_copy(ones_v, sh_deg.at[dst_v.at[j]], add=True)
            return carry

        lax.fori_loop(0, KB, step, 0)
        plsc.subcore_barrier()

        r0 = s * RPT

        @pl.when(c == 0)
        def _():
            pltpu.sync_copy(sh_agg.at[pl.ds(r0, RPT)],
                            agg0_hbm.at[pl.ds(r0, RPT)])
            if with_deg:
                pltpu.sync_copy(sh_deg.at[pl.ds(r0, RPT)],
                                deg0_hbm.at[pl.ds(r0, RPT)])

        @pl.when(c == 1)
        def _():
            pltpu.sync_copy(sh_agg.at[pl.ds(r0, RPT)],
                            agg1_hbm.at[pl.ds(r0, RPT)])
            if with_deg:
                pltpu.sync_copy(sh_deg.at[pl.ds(r0, RPT)],
                                deg1_hbm.at[pl.ds(r0, RPT)])

    k = functools.partial(
        pl.kernel, mesh=mesh, out_type=out_type, scratch_types=scratch,
        compiler_params=pltpu.CompilerParams(use_tc_tiling_on_sc=False),
    )(body)
    if with_deg:
        return k(table, srcw, dstw, zagg, zdeg, ones16)
    return k(table, srcw, dstw, zagg)


def _h_body(s1_ref, a0_ref, a1_ref, d0_ref, d1_ref, b_ref, h_ref):
    deg = d0_ref[:, 0:1] + d1_ref[:, 0:1]
    recip = 1.0 / jnp.maximum(deg, 1.0)
    agg = a0_ref[...] + a1_ref[...]
    h_ref[...] = jnp.maximum(s1_ref[...] + agg * recip + b_ref[...], 0.0)


def _h_combine(s1, agg0, agg1, deg0, deg1, b1):
    return pl.pallas_call(
        _h_body,
        grid=(N // ROWB,),
        in_specs=[pl.BlockSpec((ROWB, H), lambda i: (i, 0)),
                  pl.BlockSpec((ROWB, H), lambda i: (i, 0)),
                  pl.BlockSpec((ROWB, H), lambda i: (i, 0)),
                  pl.BlockSpec((ROWB, 16), lambda i: (i, 0)),
                  pl.BlockSpec((ROWB, 16), lambda i: (i, 0)),
                  pl.BlockSpec((1, H), lambda i: (0, 0))],
        out_specs=pl.BlockSpec((ROWB, H), lambda i: (i, 0)),
        out_shape=jax.ShapeDtypeStruct((N, H), jnp.float32),
    )(s1, agg0, agg1, deg0, deg1, b1.reshape(1, H))


def _mll_body(h_ref, a0_ref, a1_ref, d0_ref, d1_ref,
              wsm_ref, wnm_ref, bm_ref, wsl_ref, wnl_ref, bl_ref, eps_ref,
              mu_ref, ls_ref, z_ref):
    deg = d0_ref[:, 0:1] + d1_ref[:, 0:1]
    recip = 1.0 / jnp.maximum(deg, 1.0)
    m = (a0_ref[...] + a1_ref[...]) * recip
    h = h_ref[...]
    mu = (jnp.dot(h, wsm_ref[...], preferred_element_type=jnp.float32)
          + jnp.dot(m, wnm_ref[...], preferred_element_type=jnp.float32)
          + bm_ref[...])
    ls = (jnp.dot(h, wsl_ref[...], preferred_element_type=jnp.float32)
          + jnp.dot(m, wnl_ref[...], preferred_element_type=jnp.float32)
          + bl_ref[...])
    mu_ref[...] = mu
    ls_ref[...] = ls
    z_ref[...] = mu + eps_ref[...] * jnp.exp(ls)


def _mu_logstd_z(h, agg0, agg1, deg0, deg1, Wsm, Wnm, bm, Wsl, Wnl, bl, eps):
    row = lambda i: (i, 0)
    fix = lambda i: (0, 0)
    return pl.pallas_call(
        _mll_body,
        grid=(N // ROWB,),
        in_specs=[pl.BlockSpec((ROWB, H), row),
                  pl.BlockSpec((ROWB, H), row),
                  pl.BlockSpec((ROWB, H), row),
                  pl.BlockSpec((ROWB, 16), row),
                  pl.BlockSpec((ROWB, 16), row),
                  pl.BlockSpec((H, H), fix),
                  pl.BlockSpec((H, H), fix),
                  pl.BlockSpec((1, H), fix),
                  pl.BlockSpec((H, H), fix),
                  pl.BlockSpec((H, H), fix),
                  pl.BlockSpec((1, H), fix),
                  pl.BlockSpec((ROWB, H), row)],
        out_specs=[pl.BlockSpec((ROWB, H), row)] * 3,
        out_shape=[jax.ShapeDtypeStruct((N, H), jnp.float32)] * 3,
    )(h, agg0, agg1, deg0, deg1, Wsm, Wnm, bm.reshape(1, H),
      Wsl, Wnl, bl.reshape(1, H), eps)


def _dec_body(zi_ref, zj_ref, o_ref):
    o_ref[...] = jax.nn.sigmoid(
        jnp.dot(zi_ref[...], zj_ref[...], preferred_element_type=jnp.float32))


def _decode(z, zT):
    return pl.pallas_call(
        _dec_body,
        grid=(N // DEC_BR,),
        in_specs=[pl.BlockSpec((DEC_BR, H), lambda i: (i, 0)),
                  pl.BlockSpec((H, N), lambda i: (0, 0))],
        out_specs=pl.BlockSpec((DEC_BR, N), lambda i: (i, 0)),
        out_shape=jax.ShapeDtypeStruct((N, N), jnp.float32),
    )(z, zT)


def kernel(n_feats, edge_index, Ws1, Wn1, b1, Wsm, Wnm, bm, Wsl, Wnl, bl):
    src = edge_index[0]
    dst = edge_index[1]
    pad = EPAD - E
    srcw = jnp.concatenate(
        [src, jnp.zeros((pad,), jnp.int32)]).reshape(NW, KB, 128)
    dstw = jnp.concatenate(
        [dst, jnp.full((pad,), N, jnp.int32)]).reshape(NW, KB, 128)
    zagg = jnp.zeros((NPAD, H), jnp.float32)
    zdeg = jnp.zeros((NPAD, 16), jnp.float32)
    ones16 = jnp.ones((128, 16), jnp.float32)

    sp = _matmul(n_feats, jnp.concatenate([Ws1, Wn1], axis=1))
    s1 = sp[:, :H]
    p = sp[:, H:]

    agg0, agg1, deg0, deg1 = _sc_aggregate(
        p, srcw, dstw, zagg, zdeg, ones16, True)
    h = _h_combine(s1, agg0, agg1, deg0, deg1, b1)

    ah0, ah1 = _sc_aggregate(h, srcw, dstw, zagg, zdeg, ones16, False)

    eps = jax.random.normal(jax.random.key(42), (N, H), dtype=jnp.float32)
    mu, logstd, z = _mu_logstd_z(
        h, ah0, ah1, deg0, deg1, Wsm, Wnm, bm, Wsl, Wnl, bl, eps)

    adj = _decode(z, z.T)
    return adj, mu, logstd


# speedup vs baseline: 8.8838x; 1.1962x over previous
import functools

import jax
import jax.numpy as jnp
from jax import lax
from jax.experimental import pallas as pl
from jax.experimental.pallas import tpu as pltpu
from jax.experimental.pallas import tpu_sc as plsc

N = 10000
E = 320000
D = 128
H = 32

NW = 32
KB = 79
EPW = KB * 128
EPAD = NW * EPW
NPAD = 10112
RPT = NPAD // 16

ROWB = 2000
DEC_BR = 200


def _mm_body(x_ref, w_ref, o_ref):
    o_ref[...] = jnp.dot(x_ref[...], w_ref[...],
                         preferred_element_type=jnp.float32)


def _matmul(x, w):
    n, k = x.shape
    m = w.shape[1]
    return pl.pallas_call(
        _mm_body,
        grid=(n // ROWB,),
        in_specs=[pl.BlockSpec((ROWB, k), lambda i: (i, 0)),
                  pl.BlockSpec((k, m), lambda i: (0, 0))],
        out_specs=pl.BlockSpec((ROWB, m), lambda i: (i, 0)),
        out_shape=jax.ShapeDtypeStruct((n, m), jnp.float32),
    )(x, w)


def _sc_aggregate(table, srcw, dstw, zagg, zdeg, ones16, with_deg):
    mesh = plsc.VectorSubcoreMesh(core_axis_name="c", subcore_axis_name="s")

    out_type = [jax.ShapeDtypeStruct((NPAD, H), jnp.float32),
                jax.ShapeDtypeStruct((NPAD, H), jnp.float32)]
    scratch = [pltpu.VMEM((KB, 128), jnp.int32),
               pltpu.VMEM((KB, 128), jnp.int32),
               pltpu.VMEM((128, H), jnp.float32),
               pltpu.VMEM((128, H), jnp.float32),
               pltpu.VMEM_SHARED((NPAD, H), jnp.float32),
               pltpu.SemaphoreType.DMA,
               pltpu.SemaphoreType.DMA]
    if with_deg:
        out_type += [jax.ShapeDtypeStruct((NPAD, 16), jnp.float32),
                     jax.ShapeDtypeStruct((NPAD, 16), jnp.float32)]
        scratch += [pltpu.VMEM((128, 16), jnp.float32),
                    pltpu.VMEM_SHARED((NPAD, 16), jnp.float32)]

    def body(*refs):
        if with_deg:
            (tab_hbm, src_hbm, dst_hbm, zagg_hbm, zdeg_hbm, ones_hbm,
             agg0_hbm, agg1_hbm, deg0_hbm, deg1_hbm,
             src_v, dst_v, rows_a, rows_b, sh_agg, sem_a, sem_b,
             ones_v, sh_deg) = refs
        else:
            (tab_hbm, src_hbm, dst_hbm, zagg_hbm,
             agg0_hbm, agg1_hbm,
             src_v, dst_v, rows_a, rows_b, sh_agg, sem_a, sem_b) = refs
        c = lax.axis_index("c")
        s = lax.axis_index("s")
        w = s * 2 + c

        @pl.when(s == 0)
        def _():
            pltpu.sync_copy(zagg_hbm, sh_agg)
            if with_deg:
                pltpu.sync_copy(zdeg_hbm, sh_deg)

        pltpu.sync_copy(src_hbm.at[w], src_v)
        pltpu.sync_copy(dst_hbm.at[w], dst_v)
        if with_deg:
            pltpu.sync_copy(ones_hbm, ones_v)
        plsc.subcore_barrier()

        def gather(j, rows, sem):
            pltpu.async_copy(tab_hbm.at[src_v.at[j]], rows, sem)

        def drain_scatter(j, rows, sem):
            pltpu.make_async_copy(tab_hbm.at[src_v.at[j]], rows, sem).wait()
            pltpu.sync_copy(rows, sh_agg.at[dst_v.at[j]], add=True)
            if with_deg:
                pltpu.sync_copy(ones_v, sh_deg.at[dst_v.at[j]], add=True)

        gather(0, rows_a, sem_a)

        def step(jj, carry):
            j = jj * 2
            gather(j + 1, rows_b, sem_b)
            drain_scatter(j, rows_a, sem_a)
            gather(j + 2, rows_a, sem_a)
            drain_scatter(j + 1, rows_b, sem_b)
            return carry

        lax.fori_loop(0, (KB - 1) // 2, step, 0)
        drain_scatter(KB - 1, rows_a, sem_a)
        plsc.subcore_barrier()

        r0 = s * RPT

        @pl.when(c == 0)
        def _():
            pltpu.sync_copy(sh_agg.at[pl.ds(r0, RPT)],
                            agg0_hbm.at[pl.ds(r0, RPT)])
            if with_deg:
                pltpu.sync_copy(sh_deg.at[pl.ds(r0, RPT)],
                                deg0_hbm.at[pl.ds(r0, RPT)])

        @pl.when(c == 1)
        def _():
            pltpu.sync_copy(sh_agg.at[pl.ds(r0, RPT)],
                            agg1_hbm.at[pl.ds(r0, RPT)])
            if with_deg:
                pltpu.sync_copy(sh_deg.at[pl.ds(r0, RPT)],
                                deg1_hbm.at[pl.ds(r0, RPT)])

    k = functools.partial(
        pl.kernel, mesh=mesh, out_type=out_type, scratch_types=scratch,
        compiler_params=pltpu.CompilerParams(use_tc_tiling_on_sc=False),
    )(body)
    if with_deg:
        return k(table, srcw, dstw, zagg, zdeg, ones16)
    return k(table, srcw, dstw, zagg)


def _h_body(s1_ref, a0_ref, a1_ref, d0_ref, d1_ref, b_ref, h_ref):
    deg = d0_ref[:, 0:1] + d1_ref[:, 0:1]
    recip = 1.0 / jnp.maximum(deg, 1.0)
    agg = a0_ref[...] + a1_ref[...]
    h_ref[...] = jnp.maximum(s1_ref[...] + agg * recip + b_ref[...], 0.0)


def _h_combine(s1, agg0, agg1, deg0, deg1, b1):
    return pl.pallas_call(
        _h_body,
        grid=(N // ROWB,),
        in_specs=[pl.BlockSpec((ROWB, H), lambda i: (i, 0)),
                  pl.BlockSpec((ROWB, H), lambda i: (i, 0)),
                  pl.BlockSpec((ROWB, H), lambda i: (i, 0)),
                  pl.BlockSpec((ROWB, 16), lambda i: (i, 0)),
                  pl.BlockSpec((ROWB, 16), lambda i: (i, 0)),
                  pl.BlockSpec((1, H), lambda i: (0, 0))],
        out_specs=pl.BlockSpec((ROWB, H), lambda i: (i, 0)),
        out_shape=jax.ShapeDtypeStruct((N, H), jnp.float32),
    )(s1, agg0, agg1, deg0, deg1, b1.reshape(1, H))


def _mll_body(h_ref, a0_ref, a1_ref, d0_ref, d1_ref,
              wsm_ref, wnm_ref, bm_ref, wsl_ref, wnl_ref, bl_ref, eps_ref,
              mu_ref, ls_ref, z_ref):
    deg = d0_ref[:, 0:1] + d1_ref[:, 0:1]
    recip = 1.0 / jnp.maximum(deg, 1.0)
    m = (a0_ref[...] + a1_ref[...]) * recip
    h = h_ref[...]
    mu = (jnp.dot(h, wsm_ref[...], preferred_element_type=jnp.float32)
          + jnp.dot(m, wnm_ref[...], preferred_element_type=jnp.float32)
          + bm_ref[...])
    ls = (jnp.dot(h, wsl_ref[...], preferred_element_type=jnp.float32)
          + jnp.dot(m, wnl_ref[...], preferred_element_type=jnp.float32)
          + bl_ref[...])
    mu_ref[...] = mu
    ls_ref[...] = ls
    z_ref[...] = mu + eps_ref[...] * jnp.exp(ls)


def _mu_logstd_z(h, agg0, agg1, deg0, deg1, Wsm, Wnm, bm, Wsl, Wnl, bl, eps):
    row = lambda i: (i, 0)
    fix = lambda i: (0, 0)
    return pl.pallas_call(
        _mll_body,
        grid=(N // ROWB,),
        in_specs=[pl.BlockSpec((ROWB, H), row),
                  pl.BlockSpec((ROWB, H), row),
                  pl.BlockSpec((ROWB, H), row),
                  pl.BlockSpec((ROWB, 16), row),
                  pl.BlockSpec((ROWB, 16), row),
                  pl.BlockSpec((H, H), fix),
                  pl.BlockSpec((H, H), fix),
                  pl.BlockSpec((1, H), fix),
                  pl.BlockSpec((H, H), fix),
                  pl.BlockSpec((H, H), fix),
                  pl.BlockSpec((1, H), fix),
                  pl.BlockSpec((ROWB, H), row)],
        out_specs=[pl.BlockSpec((ROWB, H), row)] * 3,
        out_shape=[jax.ShapeDtypeStruct((N, H), jnp.float32)] * 3,
    )(h, agg0, agg1, deg0, deg1, Wsm, Wnm, bm.reshape(1, H),
      Wsl, Wnl, bl.reshape(1, H), eps)


def _dec_body(zi_ref, zj_ref, o_ref):
    o_ref[...] = jax.nn.sigmoid(
        jnp.dot(zi_ref[...], zj_ref[...], preferred_element_type=jnp.float32))


def _decode(z, zT):
    return pl.pallas_call(
        _dec_body,
        grid=(N // DEC_BR,),
        in_specs=[pl.BlockSpec((DEC_BR, H), lambda i: (i, 0)),
                  pl.BlockSpec((H, N), lambda i: (0, 0))],
        out_specs=pl.BlockSpec((DEC_BR, N), lambda i: (i, 0)),
        out_shape=jax.ShapeDtypeStruct((N, N), jnp.float32),
    )(z, zT)


def kernel(n_feats, edge_index, Ws1, Wn1, b1, Wsm, Wnm, bm, Wsl, Wnl, bl):
    src = edge_index[0]
    dst = edge_index[1]
    pad = EPAD - E
    srcw = jnp.concatenate(
        [src, jnp.zeros((pad,), jnp.int32)]).reshape(NW, KB, 128)
    dstw = jnp.concatenate(
        [dst, jnp.full((pad,), N, jnp.int32)]).reshape(NW, KB, 128)
    zagg = jnp.zeros((NPAD, H), jnp.float32)
    zdeg = jnp.zeros((NPAD, 16), jnp.float32)
    ones16 = jnp.ones((128, 16), jnp.float32)

    sp = _matmul(n_feats, jnp.concatenate([Ws1, Wn1], axis=1))
    s1 = sp[:, :H]
    p = sp[:, H:]

    agg0, agg1, deg0, deg1 = _sc_aggregate(
        p, srcw, dstw, zagg, zdeg, ones16, True)
    h = _h_combine(s1, agg0, agg1, deg0, deg1, b1)

    ah0, ah1 = _sc_aggregate(h, srcw, dstw, zagg, zdeg, ones16, False)

    eps = jax.random.normal(jax.random.key(42), (N, H), dtype=jnp.float32)
    mu, logstd, z = _mu_logstd_z(
        h, ah0, ah1, deg0, deg1, Wsm, Wnm, bm, Wsl, Wnl, bl, eps)

    adj = _decode(z, z.T)
    return adj, mu, logstd
